# baseline (speedup 1.0000x reference)
import functools

import jax
import jax.numpy as jnp
from jax import lax
from jax.experimental import pallas as pl
from jax.experimental.pallas import tpu as pltpu, tpu_sc as plsc

B = 16384
D = 64
NC = 2
NS = 16
NW = NC * NS
NCOL = 7813
CPW = 245
MCAP = 768
NCHK = MCAP // 128


def _stream_pass(is_item):
    def body(*refs):
        if is_item:
            (tabr, idsr, staging, out, ids_v, cb0, cb1, mids, mbuf, mbs2d,
             colmatch, ebuf, ubuf, wide, sem0, sem1, semg) = refs
        else:
            (tabr, idsr, staging, ids_v, cb0, cb1, mids, mbuf, mbs2d,
             colmatch, ebuf, wide, sem0, sem1, semg) = refs

        wid = lax.axis_index("s") * NC + lax.axis_index("c")
        ncols = jnp.where(wid < NW - 1, CPW, NCOL - (NW - 1) * CPW)
        wstart = wid * CPW
        lo = wstart * 128
        hi = lo + ncols * 128
        lane = lax.iota(jnp.int32, 16)
        zero16 = jnp.zeros((16,), jnp.int32)

        pltpu.sync_copy(idsr, ids_v)

        neg1 = jnp.full((16,), -1, jnp.int32)
        for r in range(NCHK):
            for k in range(8):
                mbs2d[r, pl.ds(k * 16, 16)] = neg1

        def scan_step(i, cnt):
            r = i // 8
            k = i % 8
            idv = ids_v[r, pl.ds(k * 16, 16)]
            m = (idv >= lo) & (idv < hi)
            plsc.store_compressed(mids.at[pl.ds(cnt, 16)], idv, mask=m)
            plsc.store_compressed(mbuf.at[pl.ds(cnt, 16)], i * 16 + lane,
                                  mask=m)
            cnt = cnt + plsc.all_reduce_population_count(m)[0]
            return jnp.minimum(cnt, MCAP - 16)

        cnt = lax.fori_loop(0, B // 16, scan_step, jnp.int32(0), unroll=8)
        cnt16 = (cnt + 15) // 16

        def fire(c, buf, sem):
            return pltpu.async_copy(
                tabr.at[:, pl.ds((wstart + c) * 128, 128)], buf, sem)

        @pl.when(0 < ncols)
        def _():
            fire(0, cb0, sem0)

        @pl.when(1 < ncols)
        def _():
            fire(1, cb1, sem1)

        def process_col(c, buf, sem, cm):
            pltpu.make_async_copy(
                tabr.at[:, pl.ds(0, 128)], buf, sem).wait()

            col_abs = wstart + c

            def rescan(j, cc):
                idv = mids[pl.ds(j * 16, 16)]
                mcol = ((idv >> 7) == col_abs) & (j * 16 + lane < cnt)
                plsc.store_compressed(colmatch.at[pl.ds(cc, 16)],
                                      j * 16 + lane, mask=mcol)
                return cc + plsc.all_reduce_population_count(mcol)[0]

            ccount = lax.fori_loop(0, MCAP // 16, rescan, jnp.int32(0),
                                   unroll=4)

            def extract(s, cm_in):
                pos = plsc.load_gather(colmatch, [jnp.broadcast_to(s, (16,))])
                idv = plsc.load_gather(mids, [pos])
                bv = plsc.load_gather(mbuf, [pos])
                lane_in_col = idv & 127
                cm_safe = jnp.minimum(cm_in, MCAP - 1)
                erow = cm_safe >> 1
                eoff = (cm_safe & 1) * 64
                for j in range(D // 16):
                    vals = plsc.load_gather(
                        buf, [j * 16 + lane, lane_in_col])
                    ebuf[erow, pl.ds(eoff + j * 16, 16)] = vals
                plsc.store_scatter(
                    mbs2d,
                    [jnp.broadcast_to(cm_safe >> 7, (16,)),
                     jnp.broadcast_to(cm_safe & 127, (16,))],
                    bv, mask=lane == 0)
                return cm_in + 1

            return lax.fori_loop(0, ccount, extract, cm)

        def superstep(s, cm):
            c0 = s * 2
            c1 = s * 2 + 1
            cm = lax.cond(c0 < ncols,
                          lambda x: process_col(c0, cb0, sem0, x),
                          lambda x: x, cm)

            @pl.when(c0 + 2 < ncols)
            def _():
                fire(c0 + 2, cb0, sem0)

            cm = lax.cond(c1 < ncols,
                          lambda x: process_col(c1, cb1, sem1, x),
                          lambda x: x, cm)

            @pl.when(c1 + 2 < ncols)
            def _():
                fire(c1 + 2, cb1, sem1)

            return cm

        lax.fori_loop(0, (CPW + 1) // 2, superstep, jnp.int32(0))

        if not is_item:
            for k in range(NCHK):
                def widen(li, carry, k=k):
                    m = k * 128 + li
                    for j in range(D // 16):
                        wide[li, pl.ds(j * 16, 16)] = \
                            ebuf[m >> 1, pl.ds((m & 1) * 64 + j * 16, 16)]
                    return carry

                lax.fori_loop(0, 128, widen, 0)
                pltpu.async_copy(
                    wide,
                    staging.at[plsc.Indices(mbs2d.at[k], ignored_value=-1)],
                    semg).wait()
        else:
            perms = [(lane + sh) & 15 for sh in (8, 4, 2, 1)]
            for k in range(NCHK):
                pltpu.async_copy(
                    staging.at[plsc.Indices(mbs2d.at[k], ignored_value=-1)],
                    ubuf, semg).wait()

                def group(g, carry, k=k):
                    acc = jnp.zeros((16,), jnp.float32)
                    for t in range(16):
                        li = g * 16 + t
                        erow = (k * 128 + li) >> 1
                        eoff = (t & 1) * 64
                        p = (ebuf[erow, pl.ds(eoff, 16)] *
                             ubuf[li, pl.ds(0, 16)])
                        for j in range(1, D // 16):
                            p += (ebuf[erow, pl.ds(eoff + j * 16, 16)] *
                                  ubuf[li, pl.ds(j * 16, 16)])
                        for perm in perms:
                            p = p + p.at[perm].get(mode="promise_in_bounds")
                        acc = jnp.where(lane == t, p, acc)
                    plsc.store_scatter(wide, [g * 16 + lane, zero16], acc)
                    return carry

                lax.fori_loop(0, 8, group, 0)
                pltpu.async_copy(
                    wide,
                    out.at[plsc.Indices(mbs2d.at[k], ignored_value=-1)],
                    semg).wait()

    return body


def _make_kernel(is_item):
    mesh = plsc.VectorSubcoreMesh(core_axis_name="c", subcore_axis_name="s")
    scratch = [
        pltpu.VMEM((B // 128, 128), jnp.int32),
        pltpu.VMEM((D, 128), jnp.float32),
        pltpu.VMEM((D, 128), jnp.float32),
        pltpu.VMEM((MCAP,), jnp.int32),
        pltpu.VMEM((MCAP,), jnp.int32),
        pltpu.VMEM((NCHK, 128), jnp.int32),
        pltpu.VMEM((MCAP,), jnp.int32),
        pltpu.VMEM((MCAP // 2, 2 * D), jnp.float32),
    ]
    if is_item:
        scratch += [pltpu.VMEM((128, 128), jnp.float32)]
    scratch += [pltpu.VMEM((128, 128), jnp.float32)]
    scratch += [pltpu.SemaphoreType.DMA, pltpu.SemaphoreType.DMA,
                pltpu.SemaphoreType.DMA]
    out_type = jax.ShapeDtypeStruct((B, 128), jnp.float32)
    return functools.partial(
        pl.kernel, mesh=mesh, out_type=out_type, scratch_types=scratch,
        compiler_params=pltpu.CompilerParams(needs_layout_passes=False),
    )(_stream_pass(is_item))


@jax.jit
def kernel(user_ids, item_ids, user_table, item_table):
    ut_t = user_table.T
    it_t = item_table.T
    uids2d = user_ids.reshape(B // 128, 128)
    iids2d = item_ids.reshape(B // 128, 128)
    staging = _make_kernel(False)(ut_t, uids2d)
    out2d = _make_kernel(True)(it_t, iids2d, staging)
    return out2d[:, 0]


# speedup vs baseline: 1.2159x; 1.2159x over previous
import functools

import jax
import jax.numpy as jnp
from jax import lax
from jax.experimental import pallas as pl
from jax.experimental.pallas import tpu as pltpu, tpu_sc as plsc

B = 16384
D = 64
NC = 2
NS = 16
NW = NC * NS
NCOL = 7813
CPW = 245
MCAP = 768
NCHK = MCAP // 128


def _stream_pass(is_item):
    def body(*refs):
        if is_item:
            (tabr, idsr, staging, out, ids_v, cb0, cb1, mids, mbuf, mbs2d,
             hist2d, laneptr, colptr, sids, sbuf,
             ebuf, ubuf, wide, sem0, sem1, semg) = refs
        else:
            (tabr, idsr, staging, ids_v, cb0, cb1, mids, mbuf, mbs2d,
             hist2d, laneptr, colptr, sids, sbuf,
             ebuf, wide, sem0, sem1, semg) = refs

        wid = lax.axis_index("s") * NC + lax.axis_index("c")
        ncols = jnp.where(wid < NW - 1, CPW, NCOL - (NW - 1) * CPW)
        wstart = wid * CPW
        lo = wstart * 128
        hi = lo + ncols * 128
        lane = lax.iota(jnp.int32, 16)
        zero16 = jnp.zeros((16,), jnp.int32)

        pltpu.sync_copy(idsr, ids_v)

        neg1 = jnp.full((16,), -1, jnp.int32)
        for r in range(NCHK):
            for k in range(8):
                mbs2d[r, pl.ds(k * 16, 16)] = neg1

        def scan_step(i, cnt):
            r = i // 8
            k = i % 8
            idv = ids_v[r, pl.ds(k * 16, 16)]
            m = (idv >= lo) & (idv < hi)
            plsc.store_compressed(mids.at[pl.ds(cnt, 16)], idv, mask=m)
            plsc.store_compressed(mbuf.at[pl.ds(cnt, 16)], i * 16 + lane,
                                  mask=m)
            cnt = cnt + plsc.all_reduce_population_count(m)[0]
            return jnp.minimum(cnt, MCAP - 16)

        cnt = lax.fori_loop(0, B // 16, scan_step, jnp.int32(0), unroll=8)

        for k in range(256 // 16):
            hist2d_zero = jnp.zeros((16,), jnp.int32)
            for r in range(16):
                hist2d[r, pl.ds(k * 16, 16)] = hist2d_zero

        def hist_step(j, carry):
            idv = mids[pl.ds(j * 16, 16)]
            valid = j * 16 + lane < cnt
            colv = jnp.where(valid, (idv >> 7) - wstart, 255)
            base = plsc.load_gather(hist2d, [lane, colv])
            plsc.store_scatter(hist2d, [lane, colv], base + 1)
            return carry

        lax.fori_loop(0, MCAP // 16, hist_step, 0, unroll=4)

        def prefix_step(k, running):
            tot = hist2d[0, pl.ds(k * 16, 16)]
            for r in range(1, 16):
                tot = tot + hist2d[r, pl.ds(k * 16, 16)]
            csum = plsc.cumsum(tot)
            excl = running + csum - tot
            colptr[pl.ds(k * 16, 16)] = excl
            acc = excl
            for r in range(16):
                laneptr[r, pl.ds(k * 16, 16)] = acc
                acc = acc + hist2d[r, pl.ds(k * 16, 16)]
            return running + csum[15]

        lax.fori_loop(0, 256 // 16, prefix_step, jnp.int32(0))

        def sort_step(j, carry):
            idv = mids[pl.ds(j * 16, 16)]
            bv = mbuf[pl.ds(j * 16, 16)]
            valid = j * 16 + lane < cnt
            colv = jnp.where(valid, (idv >> 7) - wstart, 255)
            off = plsc.load_gather(laneptr, [lane, colv])
            plsc.store_scatter(sids, [off], idv, mask=valid)
            plsc.store_scatter(sbuf, [off], bv, mask=valid)
            plsc.store_scatter(laneptr, [lane, colv], off + 1)
            return carry

        lax.fori_loop(0, MCAP // 16, sort_step, 0, unroll=4)

        def fire(c, buf, sem):
            return pltpu.async_copy(
                tabr.at[:, pl.ds((wstart + c) * 128, 128)], buf, sem)

        @pl.when(0 < ncols)
        def _():
            fire(0, cb0, sem0)

        @pl.when(1 < ncols)
        def _():
            fire(1, cb1, sem1)

        def process_col(c, buf, sem, cm):
            pltpu.make_async_copy(
                tabr.at[:, pl.ds(0, 128)], buf, sem).wait()

            bounds = plsc.load_gather(
                colptr, [c + (lane & 1)])
            s0 = bounds[0]
            s1 = bounds[1]

            def extract(s, cm_in):
                sv = jnp.broadcast_to(s, (16,))
                idv = plsc.load_gather(sids, [sv])
                bv = plsc.load_gather(sbuf, [sv])
                lane_in_col = idv & 127
                cm_safe = jnp.minimum(s, MCAP - 1)
                erow = cm_safe >> 1
                eoff = (cm_safe & 1) * 64
                for j in range(D // 16):
                    vals = plsc.load_gather(
                        buf, [j * 16 + lane, lane_in_col])
                    ebuf[erow, pl.ds(eoff + j * 16, 16)] = vals
                plsc.store_scatter(
                    mbs2d,
                    [jnp.broadcast_to(cm_safe >> 7, (16,)),
                     jnp.broadcast_to(cm_safe & 127, (16,))],
                    bv, mask=lane == 0)
                return cm_in

            return lax.fori_loop(s0, s1, extract, cm)

        def superstep(s, cm):
            c0 = s * 2
            c1 = s * 2 + 1
            cm = lax.cond(c0 < ncols,
                          lambda x: process_col(c0, cb0, sem0, x),
                          lambda x: x, cm)

            @pl.when(c0 + 2 < ncols)
            def _():
                fire(c0 + 2, cb0, sem0)

            cm = lax.cond(c1 < ncols,
                          lambda x: process_col(c1, cb1, sem1, x),
                          lambda x: x, cm)

            @pl.when(c1 + 2 < ncols)
            def _():
                fire(c1 + 2, cb1, sem1)

            return cm

        lax.fori_loop(0, (CPW + 1) // 2, superstep, jnp.int32(0))

        if not is_item:
            for k in range(NCHK):
                def widen(li, carry, k=k):
                    m = k * 128 + li
                    for j in range(D // 16):
                        wide[li, pl.ds(j * 16, 16)] = \
                            ebuf[m >> 1, pl.ds((m & 1) * 64 + j * 16, 16)]
                    return carry

                lax.fori_loop(0, 128, widen, 0)
                pltpu.async_copy(
                    wide,
                    staging.at[plsc.Indices(mbs2d.at[k], ignored_value=-1)],
                    semg).wait()
        else:
            perms = [(lane + sh) & 15 for sh in (8, 4, 2, 1)]
            for k in range(NCHK):
                pltpu.async_copy(
                    staging.at[plsc.Indices(mbs2d.at[k], ignored_value=-1)],
                    ubuf, semg).wait()

                def group(g, carry, k=k):
                    acc = jnp.zeros((16,), jnp.float32)
                    for t in range(16):
                        li = g * 16 + t
                        erow = (k * 128 + li) >> 1
                        eoff = (t & 1) * 64
                        p = (ebuf[erow, pl.ds(eoff, 16)] *
                             ubuf[li, pl.ds(0, 16)])
                        for j in range(1, D // 16):
                            p += (ebuf[erow, pl.ds(eoff + j * 16, 16)] *
                                  ubuf[li, pl.ds(j * 16, 16)])
                        for perm in perms:
                            p = p + p.at[perm].get(mode="promise_in_bounds")
                        acc = jnp.where(lane == t, p, acc)
                    plsc.store_scatter(wide, [g * 16 + lane, zero16], acc)
                    return carry

                lax.fori_loop(0, 8, group, 0)
                pltpu.async_copy(
                    wide,
                    out.at[plsc.Indices(mbs2d.at[k], ignored_value=-1)],
                    semg).wait()

    return body


def _make_kernel(is_item):
    mesh = plsc.VectorSubcoreMesh(core_axis_name="c", subcore_axis_name="s")
    scratch = [
        pltpu.VMEM((B // 128, 128), jnp.int32),
        pltpu.VMEM((D, 128), jnp.float32),
        pltpu.VMEM((D, 128), jnp.float32),
        pltpu.VMEM((MCAP,), jnp.int32),
        pltpu.VMEM((MCAP,), jnp.int32),
        pltpu.VMEM((NCHK, 128), jnp.int32),
        pltpu.VMEM((16, 256), jnp.int32),
        pltpu.VMEM((16, 256), jnp.int32),
        pltpu.VMEM((272,), jnp.int32),
        pltpu.VMEM((MCAP,), jnp.int32),
        pltpu.VMEM((MCAP,), jnp.int32),
        pltpu.VMEM((MCAP // 2, 2 * D), jnp.float32),
    ]
    if is_item:
        scratch += [pltpu.VMEM((128, 128), jnp.float32)]
    scratch += [pltpu.VMEM((128, 128), jnp.float32)]
    scratch += [pltpu.SemaphoreType.DMA, pltpu.SemaphoreType.DMA,
                pltpu.SemaphoreType.DMA]
    out_type = jax.ShapeDtypeStruct((B, 128), jnp.float32)
    return functools.partial(
        pl.kernel, mesh=mesh, out_type=out_type, scratch_types=scratch,
        compiler_params=pltpu.CompilerParams(needs_layout_passes=False),
    )(_stream_pass(is_item))


@jax.jit
def kernel(user_ids, item_ids, user_table, item_table):
    ut_t = user_table.T
    it_t = item_table.T
    uids2d = user_ids.reshape(B // 128, 128)
    iids2d = item_ids.reshape(B // 128, 128)
    staging = _make_kernel(False)(ut_t, uids2d)
    out2d = _make_kernel(True)(it_t, iids2d, staging)
    return out2d[:, 0]


# speedup vs baseline: 1.4063x; 1.1566x over previous
import functools

import jax
import jax.numpy as jnp
from jax import lax
from jax.experimental import pallas as pl
from jax.experimental.pallas import tpu as pltpu, tpu_sc as plsc

B = 16384
D = 64
NC = 2
NS = 16
NW = NC * NS
CPW = 123
MCAP = 640
NCHK = MCAP // 128


def _stream_pass(is_item):
    def body(*refs):
        if is_item:
            (tabr, idsr, staging, out, ids_v, cb0, cb1, mids, mbuf, mbs2d,
             hist2d, laneptr, colptr, sids, sbuf,
             ebuf, ubuf, wide, sem0, sem1, semg) = refs
        else:
            (tabr, idsr, staging, ids_v, cb0, cb1, mids, mbuf, mbs2d,
             hist2d, laneptr, colptr, sids, sbuf,
             ebuf, wide, sem0, sem1, semg) = refs

        wid = lax.axis_index("s") * NC + lax.axis_index("c")
        is_last = wid == NW - 1
        cwb = jnp.where(is_last, 7, 8)
        cmask = jnp.where(is_last, 127, 255)
        ncols = jnp.where(is_last, 187, CPW)
        lo = wid * CPW * 256
        hi = jnp.where(is_last, 1000000, lo + CPW * 256)
        lane = lax.iota(jnp.int32, 16)
        zero16 = jnp.zeros((16,), jnp.int32)

        neg1 = jnp.full((16,), -1, jnp.int32)
        for r in range(NCHK):
            for k in range(8):
                mbs2d[r, pl.ds(k * 16, 16)] = neg1

        def scan_block(blk, cnt):
            pltpu.sync_copy(idsr.at[pl.ds(blk * 16, 16)], ids_v)

            def scan_step(i, cnt):
                r = i // 8
                k = i % 8
                idv = ids_v[r, pl.ds(k * 16, 16)]
                m = (idv >= lo) & (idv < hi)
                b0 = (blk * 16 + r) * 128 + k * 16
                plsc.store_compressed(mids.at[pl.ds(cnt, 16)], idv, mask=m)
                plsc.store_compressed(mbuf.at[pl.ds(cnt, 16)], b0 + lane,
                                      mask=m)
                cnt = cnt + plsc.all_reduce_population_count(m)[0]
                return jnp.minimum(cnt, MCAP - 16)

            return lax.fori_loop(0, 128, scan_step, cnt, unroll=8)

        cnt = lax.fori_loop(0, 8, scan_block, jnp.int32(0))

        for k in range(256 // 16):
            hist2d_zero = jnp.zeros((16,), jnp.int32)
            for r in range(16):
                hist2d[r, pl.ds(k * 16, 16)] = hist2d_zero

        def hist_step(j, carry):
            idv = mids[pl.ds(j * 16, 16)]
            valid = j * 16 + lane < cnt
            colv = jnp.where(valid, (idv - lo) >> cwb, 255)
            base = plsc.load_gather(hist2d, [lane, colv])
            plsc.store_scatter(hist2d, [lane, colv], base + 1)
            return carry

        lax.fori_loop(0, MCAP // 16, hist_step, 0, unroll=4)

        def prefix_step(k, running):
            tot = hist2d[0, pl.ds(k * 16, 16)]
            for r in range(1, 16):
                tot = tot + hist2d[r, pl.ds(k * 16, 16)]
            csum = plsc.cumsum(tot)
            excl = running + csum - tot
            colptr[pl.ds(k * 16, 16)] = excl
            acc = excl
            for r in range(16):
                laneptr[r, pl.ds(k * 16, 16)] = acc
                acc = acc + hist2d[r, pl.ds(k * 16, 16)]
            return running + csum[15]

        lax.fori_loop(0, 256 // 16, prefix_step, jnp.int32(0))

        def sort_step(j, carry):
            idv = mids[pl.ds(j * 16, 16)]
            bv = mbuf[pl.ds(j * 16, 16)]
            valid = j * 16 + lane < cnt
            colv = jnp.where(valid, (idv - lo) >> cwb, 255)
            off = plsc.load_gather(laneptr, [lane, colv])
            plsc.store_scatter(sids, [off], idv, mask=valid)
            plsc.store_scatter(sbuf, [off], bv, mask=valid)
            plsc.store_scatter(laneptr, [lane, colv], off + 1)
            return carry

        lax.fori_loop(0, MCAP // 16, sort_step, 0, unroll=4)

        def fire(c, buf, sem):
            base = pl.multiple_of(lo + (c << cwb), 128)

            @pl.when(jnp.logical_not(is_last))
            def _():
                pltpu.async_copy(tabr.at[:, pl.ds(base, 256)], buf, sem)

            @pl.when(is_last)
            def _():
                pltpu.async_copy(tabr.at[:, pl.ds(base, 128)],
                                 buf.at[:, pl.ds(0, 128)], sem)

        @pl.when(0 < ncols)
        def _():
            fire(0, cb0, sem0)

        @pl.when(1 < ncols)
        def _():
            fire(1, cb1, sem1)

        def process_col(c, buf, sem, cm):
            @pl.when(jnp.logical_not(is_last))
            def _():
                pltpu.make_async_copy(
                    tabr.at[:, pl.ds(0, 256)], buf, sem).wait()

            @pl.when(is_last)
            def _():
                pltpu.make_async_copy(
                    tabr.at[:, pl.ds(0, 128)],
                    buf.at[:, pl.ds(0, 128)], sem).wait()

            bounds = plsc.load_gather(
                colptr, [c + (lane & 1)])
            s0 = bounds[0]
            s1 = bounds[1]

            def extract(s, cm_in):
                sv = jnp.broadcast_to(s, (16,))
                idv = plsc.load_gather(sids, [sv])
                bv = plsc.load_gather(sbuf, [sv])
                lane_in_col = (idv - lo) & cmask
                cm_safe = jnp.minimum(s, MCAP - 1)
                erow = cm_safe >> 1
                eoff = (cm_safe & 1) * 64
                for j in range(D // 16):
                    vals = plsc.load_gather(
                        buf, [j * 16 + lane, lane_in_col])
                    ebuf[erow, pl.ds(eoff + j * 16, 16)] = vals
                plsc.store_scatter(
                    mbs2d,
                    [jnp.broadcast_to(cm_safe >> 7, (16,)),
                     jnp.broadcast_to(cm_safe & 127, (16,))],
                    bv, mask=lane == 0)
                return cm_in

            return lax.fori_loop(s0, s1, extract, cm)

        def superstep(s, cm):
            c0 = s * 2
            c1 = s * 2 + 1
            cm = lax.cond(c0 < ncols,
                          lambda x: process_col(c0, cb0, sem0, x),
                          lambda x: x, cm)

            @pl.when(c0 + 2 < ncols)
            def _():
                fire(c0 + 2, cb0, sem0)

            cm = lax.cond(c1 < ncols,
                          lambda x: process_col(c1, cb1, sem1, x),
                          lambda x: x, cm)

            @pl.when(c1 + 2 < ncols)
            def _():
                fire(c1 + 2, cb1, sem1)

            return cm

        lax.fori_loop(0, 94, superstep, jnp.int32(0))

        if not is_item:
            for k in range(NCHK):
                def widen(li, carry, k=k):
                    m = k * 128 + li
                    for j in range(D // 16):
                        wide[li, pl.ds(j * 16, 16)] = \
                            ebuf[m >> 1, pl.ds((m & 1) * 64 + j * 16, 16)]
                    return carry

                lax.fori_loop(0, 128, widen, 0)
                pltpu.async_copy(
                    wide,
                    staging.at[plsc.Indices(mbs2d.at[k], ignored_value=-1)],
                    semg).wait()
        else:
            perms = [(lane + sh) & 15 for sh in (8, 4, 2, 1)]
            for k in range(NCHK):
                pltpu.async_copy(
                    staging.at[plsc.Indices(mbs2d.at[k], ignored_value=-1)],
                    ubuf, semg).wait()

                def group(g, carry, k=k):
                    acc = jnp.zeros((16,), jnp.float32)
                    for t in range(16):
                        li = g * 16 + t
                        erow = (k * 128 + li) >> 1
                        eoff = (t & 1) * 64
                        p = (ebuf[erow, pl.ds(eoff, 16)] *
                             ubuf[li, pl.ds(0, 16)])
                        for j in range(1, D // 16):
                            p += (ebuf[erow, pl.ds(eoff + j * 16, 16)] *
                                  ubuf[li, pl.ds(j * 16, 16)])
                        for perm in perms:
                            p = p + p.at[perm].get(mode="promise_in_bounds")
                        acc = jnp.where(lane == t, p, acc)
                    plsc.store_scatter(wide, [g * 16 + lane, zero16], acc)
                    return carry

                lax.fori_loop(0, 8, group, 0)
                pltpu.async_copy(
                    wide,
                    out.at[plsc.Indices(mbs2d.at[k], ignored_value=-1)],
                    semg).wait()

    return body


def _make_kernel(is_item):
    mesh = plsc.VectorSubcoreMesh(core_axis_name="c", subcore_axis_name="s")
    scratch = [
        pltpu.VMEM((16, 128), jnp.int32),
        pltpu.VMEM((D, 256), jnp.float32),
        pltpu.VMEM((D, 256), jnp.float32),
        pltpu.VMEM((MCAP,), jnp.int32),
        pltpu.VMEM((MCAP,), jnp.int32),
        pltpu.VMEM((NCHK, 128), jnp.int32),
        pltpu.VMEM((16, 256), jnp.int32),
        pltpu.VMEM((16, 256), jnp.int32),
        pltpu.VMEM((272,), jnp.int32),
        pltpu.VMEM((MCAP,), jnp.int32),
        pltpu.VMEM((MCAP,), jnp.int32),
        pltpu.VMEM((MCAP // 2, 2 * D), jnp.float32),
    ]
    if is_item:
        scratch += [pltpu.VMEM((128, 128), jnp.float32)]
    scratch += [pltpu.VMEM((128, 128), jnp.float32)]
    scratch += [pltpu.SemaphoreType.DMA, pltpu.SemaphoreType.DMA,
                pltpu.SemaphoreType.DMA]
    out_type = jax.ShapeDtypeStruct((B, 128), jnp.float32)
    return functools.partial(
        pl.kernel, mesh=mesh, out_type=out_type, scratch_types=scratch,
        compiler_params=pltpu.CompilerParams(needs_layout_passes=False),
    )(_stream_pass(is_item))


@jax.jit
def kernel(user_ids, item_ids, user_table, item_table):
    ut_t = user_table.T
    it_t = item_table.T
    uids2d = user_ids.reshape(B // 128, 128)
    iids2d = item_ids.reshape(B // 128, 128)
    staging = _make_kernel(False)(ut_t, uids2d)
    out2d = _make_kernel(True)(it_t, iids2d, staging)
    return out2d[:, 0]


# speedup vs baseline: 1.6077x; 1.1432x over previous
import functools

import jax
import jax.numpy as jnp
from jax import lax
from jax.experimental import pallas as pl
from jax.experimental.pallas import tpu as pltpu, tpu_sc as plsc

B = 16384
D = 64
NC = 2
NS = 16
NW = NC * NS
NCOL = 7813
CPW = 245
MCAP = 640
NCHK = MCAP // 128


def _stream_pass(is_item):
    def body(*refs):
        if is_item:
            (tabr, idsr, staging, out, ids_v, cb0, cb1, cb2, cb3,
             mids, mbuf, mbs2d, hist2d, laneptr, colptr, sids, sbuf,
             ebuf, ubuf, wide, sem0, sem1, sem2, sem3, semg) = refs
        else:
            (tabr, idsr, staging, ids_v, cb0, cb1, cb2, cb3,
             mids, mbuf, mbs2d, hist2d, laneptr, colptr, sids, sbuf,
             ebuf, wide, sem0, sem1, sem2, sem3, semg) = refs
        cbufs = (cb0, cb1, cb2, cb3)
        sems = (sem0, sem1, sem2, sem3)

        wid = lax.axis_index("s") * NC + lax.axis_index("c")
        is_last = wid == NW - 1
        ncols = jnp.where(is_last, NCOL - (NW - 1) * CPW, CPW)
        lo = wid * CPW * 128
        hi = jnp.where(is_last, 1000000, lo + CPW * 128)
        lane = lax.iota(jnp.int32, 16)
        zero16 = jnp.zeros((16,), jnp.int32)

        neg1 = jnp.full((16,), -1, jnp.int32)
        for r in range(NCHK):
            for k in range(8):
                mbs2d[r, pl.ds(k * 16, 16)] = neg1

        def scan_block(blk, cnt):
            pltpu.sync_copy(idsr.at[pl.ds(blk * 16, 16)], ids_v)

            def scan_step(i, cnt):
                r = i // 8
                k = i % 8
                idv = ids_v[r, pl.ds(k * 16, 16)]
                m = (idv >= lo) & (idv < hi)
                b0 = (blk * 16 + r) * 128 + k * 16
                plsc.store_compressed(mids.at[pl.ds(cnt, 16)], idv, mask=m)
                plsc.store_compressed(mbuf.at[pl.ds(cnt, 16)], b0 + lane,
                                      mask=m)
                cnt = cnt + plsc.all_reduce_population_count(m)[0]
                return jnp.minimum(cnt, MCAP - 16)

            return lax.fori_loop(0, 128, scan_step, cnt, unroll=8)

        cnt = lax.fori_loop(0, 8, scan_block, jnp.int32(0))

        for k in range(256 // 16):
            hist2d_zero = jnp.zeros((16,), jnp.int32)
            for r in range(16):
                hist2d[r, pl.ds(k * 16, 16)] = hist2d_zero

        def hist_step(j, carry):
            idv = mids[pl.ds(j * 16, 16)]
            valid = j * 16 + lane < cnt
            colv = jnp.where(valid, (idv - lo) >> 7, 255)
            base = plsc.load_gather(hist2d, [lane, colv])
            plsc.store_scatter(hist2d, [lane, colv], base + 1)
            return carry

        lax.fori_loop(0, MCAP // 16, hist_step, 0, unroll=4)

        def prefix_step(k, running):
            tot = hist2d[0, pl.ds(k * 16, 16)]
            for r in range(1, 16):
                tot = tot + hist2d[r, pl.ds(k * 16, 16)]
            csum = plsc.cumsum(tot)
            excl = running + csum - tot
            colptr[pl.ds(k * 16, 16)] = excl
            acc = excl
            for r in range(16):
                laneptr[r, pl.ds(k * 16, 16)] = acc
                acc = acc + hist2d[r, pl.ds(k * 16, 16)]
            return running + csum[15]

        lax.fori_loop(0, 256 // 16, prefix_step, jnp.int32(0))

        def sort_step(j, carry):
            idv = mids[pl.ds(j * 16, 16)]
            bv = mbuf[pl.ds(j * 16, 16)]
            valid = j * 16 + lane < cnt
            colv = jnp.where(valid, (idv - lo) >> 7, 255)
            off = plsc.load_gather(laneptr, [lane, colv])
            plsc.store_scatter(sids, [off], idv, mask=valid)
            plsc.store_scatter(sbuf, [off], bv, mask=valid)
            plsc.store_scatter(laneptr, [lane, colv], off + 1)
            return carry

        lax.fori_loop(0, MCAP // 16, sort_step, 0, unroll=4)

        def fire(c, buf, sem):
            base = pl.multiple_of(lo + c * 128, 128)
            pltpu.async_copy(tabr.at[:, pl.ds(base, 128)], buf, sem)

        for par in range(4):
            @pl.when(par < ncols)
            def _(par=par):
                fire(par, cbufs[par], sems[par])

        def process_col(c, buf, sem, cm):
            pltpu.make_async_copy(
                tabr.at[:, pl.ds(0, 128)], buf, sem).wait()

            bounds = plsc.load_gather(
                colptr, [c + (lane & 1)])
            s0 = bounds[0]
            s1 = bounds[1]

            def extract(s, cm_in):
                sv = jnp.broadcast_to(s, (16,))
                idv = plsc.load_gather(sids, [sv])
                bv = plsc.load_gather(sbuf, [sv])
                lane_in_col = (idv - lo) & 127
                cm_safe = jnp.minimum(s, MCAP - 1)
                erow = cm_safe >> 1
                eoff = (cm_safe & 1) * 64
                for j in range(D // 16):
                    vals = plsc.load_gather(
                        buf, [j * 16 + lane, lane_in_col])
                    ebuf[erow, pl.ds(eoff + j * 16, 16)] = vals
                plsc.store_scatter(
                    mbs2d,
                    [jnp.broadcast_to(cm_safe >> 7, (16,)),
                     jnp.broadcast_to(cm_safe & 127, (16,))],
                    bv, mask=lane == 0)
                return cm_in

            return lax.fori_loop(s0, s1, extract, cm)

        def superstep(s, cm):
            for par in range(4):
                c = s * 4 + par
                cm = lax.cond(c < ncols,
                              lambda x, c=c, par=par: process_col(
                                  c, cbufs[par], sems[par], x),
                              lambda x: x, cm)

                @pl.when(c + 4 < ncols)
                def _(c=c, par=par):
                    fire(c + 4, cbufs[par], sems[par])

            return cm

        lax.fori_loop(0, (CPW + 3) // 4, superstep, jnp.int32(0))

        if not is_item:
            for k in range(NCHK):
                def widen(li, carry, k=k):
                    m = k * 128 + li
                    for j in range(D // 16):
                        wide[li, pl.ds(j * 16, 16)] = \
                            ebuf[m >> 1, pl.ds((m & 1) * 64 + j * 16, 16)]
                    return carry

                lax.fori_loop(0, 128, widen, 0)
                pltpu.async_copy(
                    wide,
                    staging.at[plsc.Indices(mbs2d.at[k], ignored_value=-1)],
                    semg).wait()
        else:
            perms = [(lane + sh) & 15 for sh in (8, 4, 2, 1)]
            for k in range(NCHK):
                pltpu.async_copy(
                    staging.at[plsc.Indices(mbs2d.at[k], ignored_value=-1)],
                    ubuf, semg).wait()

                def group(g, carry, k=k):
                    acc = jnp.zeros((16,), jnp.float32)
                    for t in range(16):
                        li = g * 16 + t
                        erow = (k * 128 + li) >> 1
                        eoff = (t & 1) * 64
                        p = (ebuf[erow, pl.ds(eoff, 16)] *
                             ubuf[li, pl.ds(0, 16)])
                        for j in range(1, D // 16):
                            p += (ebuf[erow, pl.ds(eoff + j * 16, 16)] *
                                  ubuf[li, pl.ds(j * 16, 16)])
                        for perm in perms:
                            p = p + p.at[perm].get(mode="promise_in_bounds")
                        acc = jnp.where(lane == t, p, acc)
                    plsc.store_scatter(wide, [g * 16 + lane, zero16], acc)
                    return carry

                lax.fori_loop(0, 8, group, 0)
                pltpu.async_copy(
                    wide,
                    out.at[plsc.Indices(mbs2d.at[k], ignored_value=-1)],
                    semg).wait()

    return body


def _make_kernel(is_item):
    mesh = plsc.VectorSubcoreMesh(core_axis_name="c", subcore_axis_name="s")
    scratch = [
        pltpu.VMEM((16, 128), jnp.int32),
        pltpu.VMEM((D, 128), jnp.float32),
        pltpu.VMEM((D, 128), jnp.float32),
        pltpu.VMEM((D, 128), jnp.float32),
        pltpu.VMEM((D, 128), jnp.float32),
        pltpu.VMEM((MCAP,), jnp.int32),
        pltpu.VMEM((MCAP,), jnp.int32),
        pltpu.VMEM((NCHK, 128), jnp.int32),
        pltpu.VMEM((16, 256), jnp.int32),
        pltpu.VMEM((16, 256), jnp.int32),
        pltpu.VMEM((272,), jnp.int32),
        pltpu.VMEM((MCAP,), jnp.int32),
        pltpu.VMEM((MCAP,), jnp.int32),
        pltpu.VMEM((MCAP // 2, 2 * D), jnp.float32),
    ]
    if is_item:
        scratch += [pltpu.VMEM((128, 128), jnp.float32)]
    scratch += [pltpu.VMEM((128, 128), jnp.float32)]
    scratch += [pltpu.SemaphoreType.DMA] * 5
    out_type = jax.ShapeDtypeStruct((B, 128), jnp.float32)
    return functools.partial(
        pl.kernel, mesh=mesh, out_type=out_type, scratch_types=scratch,
        compiler_params=pltpu.CompilerParams(needs_layout_passes=False),
    )(_stream_pass(is_item))


@jax.jit
def kernel(user_ids, item_ids, user_table, item_table):
    ut_t = user_table.T
    it_t = item_table.T
    uids2d = user_ids.reshape(B // 128, 128)
    iids2d = item_ids.reshape(B // 128, 128)
    staging = _make_kernel(False)(ut_t, uids2d)
    out2d = _make_kernel(True)(it_t, iids2d, staging)
    return out2d[:, 0]


# speedup vs baseline: 1.7285x; 1.0752x over previous
import functools

import jax
import jax.numpy as jnp
from jax import lax
from jax.experimental import pallas as pl
from jax.experimental.pallas import tpu as pltpu, tpu_sc as plsc

B = 16384
D = 64
NC = 2
NS = 16
NW = NC * NS
NCOL = 7813
CPW = 245
MCAP = 640
NCHK = MCAP // 128


def _stream_pass(is_item):
    def body(*refs):
        if is_item:
            (tabr, idsr, staging, out, ids_v, cb0, cb1, cb2, cb3,
             mids, mbuf, mbs2d, hist2d, laneptr, colptr, necols, sids, sbuf,
             ebuf, ubuf, wide, sem0, sem1, sem2, sem3, semg) = refs
        else:
            (tabr, idsr, staging, ids_v, cb0, cb1, cb2, cb3,
             mids, mbuf, mbs2d, hist2d, laneptr, colptr, necols, sids, sbuf,
             ebuf, wide, sem0, sem1, sem2, sem3, semg) = refs
        cbufs = (cb0, cb1, cb2, cb3)
        sems = (sem0, sem1, sem2, sem3)

        wid = lax.axis_index("s") * NC + lax.axis_index("c")
        is_last = wid == NW - 1
        ncols = jnp.where(is_last, NCOL - (NW - 1) * CPW, CPW)
        lo = wid * CPW * 128
        hi = jnp.where(is_last, 1000000, lo + CPW * 128)
        lane = lax.iota(jnp.int32, 16)
        zero16 = jnp.zeros((16,), jnp.int32)

        neg1 = jnp.full((16,), -1, jnp.int32)
        for r in range(NCHK):
            for k in range(8):
                mbs2d[r, pl.ds(k * 16, 16)] = neg1

        def scan_block(blk, cnt):
            pltpu.sync_copy(idsr.at[pl.ds(blk * 16, 16)], ids_v)

            def scan_step(i, cnt):
                r = i // 8
                k = i % 8
                idv = ids_v[r, pl.ds(k * 16, 16)]
                m = (idv >= lo) & (idv < hi)
                b0 = (blk * 16 + r) * 128 + k * 16
                plsc.store_compressed(mids.at[pl.ds(cnt, 16)], idv, mask=m)
                plsc.store_compressed(mbuf.at[pl.ds(cnt, 16)], b0 + lane,
                                      mask=m)
                cnt = cnt + plsc.all_reduce_population_count(m)[0]
                return jnp.minimum(cnt, MCAP - 16)

            return lax.fori_loop(0, 128, scan_step, cnt, unroll=8)

        cnt = lax.fori_loop(0, 8, scan_block, jnp.int32(0))

        for k in range(256 // 16):
            hist2d_zero = jnp.zeros((16,), jnp.int32)
            for r in range(16):
                hist2d[r, pl.ds(k * 16, 16)] = hist2d_zero

        def hist_step(j, carry):
            idv = mids[pl.ds(j * 16, 16)]
            valid = j * 16 + lane < cnt
            colv = jnp.where(valid, (idv - lo) >> 7, 255)
            base = plsc.load_gather(hist2d, [lane, colv])
            plsc.store_scatter(hist2d, [lane, colv], base + 1)
            return carry

        lax.fori_loop(0, MCAP // 16, hist_step, 0, unroll=4)

        def prefix_step(k, running):
            tot = hist2d[0, pl.ds(k * 16, 16)]
            for r in range(1, 16):
                tot = tot + hist2d[r, pl.ds(k * 16, 16)]
            csum = plsc.cumsum(tot)
            excl = running + csum - tot
            colptr[pl.ds(k * 16, 16)] = excl
            acc = excl
            for r in range(16):
                laneptr[r, pl.ds(k * 16, 16)] = acc
                acc = acc + hist2d[r, pl.ds(k * 16, 16)]
            return running + csum[15]

        lax.fori_loop(0, 256 // 16, prefix_step, jnp.int32(0))

        def sort_step(j, carry):
            idv = mids[pl.ds(j * 16, 16)]
            bv = mbuf[pl.ds(j * 16, 16)]
            valid = j * 16 + lane < cnt
            colv = jnp.where(valid, (idv - lo) >> 7, 255)
            off = plsc.load_gather(laneptr, [lane, colv])
            plsc.store_scatter(sids, [off], idv, mask=valid)
            plsc.store_scatter(sbuf, [off], bv, mask=valid)
            plsc.store_scatter(laneptr, [lane, colv], off + 1)
            return carry

        lax.fori_loop(0, MCAP // 16, sort_step, 0, unroll=4)

        def ne_step(k, nc):
            cur = colptr[pl.ds(k * 16, 16)]
            nxt = colptr[pl.ds(k * 16 + 1, 16)]
            colidx = k * 16 + lane
            m = (nxt > cur) & (colidx < ncols)
            plsc.store_compressed(necols.at[pl.ds(nc, 16)], colidx, mask=m)
            return nc + plsc.all_reduce_population_count(m)[0]

        nc_total = lax.fori_loop(0, 256 // 16, ne_step, jnp.int32(0))

        def nth_col(i):
            return plsc.load_gather(necols, [jnp.broadcast_to(i, (16,))])[0]

        def fire(i, buf, sem):
            base = pl.multiple_of(lo + nth_col(i) * 128, 128)
            pltpu.async_copy(tabr.at[:, pl.ds(base, 128)], buf, sem)

        for par in range(4):
            @pl.when(par < nc_total)
            def _(par=par):
                fire(par, cbufs[par], sems[par])

        def process_col(i, buf, sem, cm):
            pltpu.make_async_copy(
                tabr.at[:, pl.ds(0, 128)], buf, sem).wait()

            c = nth_col(i)
            bounds = plsc.load_gather(
                colptr, [c + (lane & 1)])
            s0 = bounds[0]
            s1 = bounds[1]

            def extract(s, cm_in):
                sv = jnp.broadcast_to(s, (16,))
                idv = plsc.load_gather(sids, [sv])
                bv = plsc.load_gather(sbuf, [sv])
                lane_in_col = (idv - lo) & 127
                cm_safe = jnp.minimum(s, MCAP - 1)
                erow = cm_safe >> 1
                eoff = (cm_safe & 1) * 64
                for j in range(D // 16):
                    vals = plsc.load_gather(
                        buf, [j * 16 + lane, lane_in_col])
                    ebuf[erow, pl.ds(eoff + j * 16, 16)] = vals
                plsc.store_scatter(
                    mbs2d,
                    [jnp.broadcast_to(cm_safe >> 7, (16,)),
                     jnp.broadcast_to(cm_safe & 127, (16,))],
                    bv, mask=lane == 0)
                return cm_in

            return lax.fori_loop(s0, s1, extract, cm)

        def superstep(s, cm):
            for par in range(4):
                i = s * 4 + par
                cm = lax.cond(i < nc_total,
                              lambda x, i=i, par=par: process_col(
                                  i, cbufs[par], sems[par], x),
                              lambda x: x, cm)

                @pl.when(i + 4 < nc_total)
                def _(i=i, par=par):
                    fire(i + 4, cbufs[par], sems[par])

            return cm

        lax.fori_loop(0, (CPW + 3) // 4, superstep, jnp.int32(0))

        if not is_item:
            for k in range(NCHK):
                def widen(li, carry, k=k):
                    m = k * 128 + li
                    for j in range(D // 16):
                        wide[li, pl.ds(j * 16, 16)] = \
                            ebuf[m >> 1, pl.ds((m & 1) * 64 + j * 16, 16)]
                    return carry

                lax.fori_loop(0, 128, widen, 0)
                pltpu.async_copy(
                    wide,
                    staging.at[plsc.Indices(mbs2d.at[k], ignored_value=-1)],
                    semg).wait()
        else:
            perms = [(lane + sh) & 15 for sh in (8, 4, 2, 1)]
            for k in range(NCHK):
                pltpu.async_copy(
                    staging.at[plsc.Indices(mbs2d.at[k], ignored_value=-1)],
                    ubuf, semg).wait()

                def group(g, carry, k=k):
                    acc = jnp.zeros((16,), jnp.float32)
                    for t in range(16):
                        li = g * 16 + t
                        erow = (k * 128 + li) >> 1
                        eoff = (t & 1) * 64
                        p = (ebuf[erow, pl.ds(eoff, 16)] *
                             ubuf[li, pl.ds(0, 16)])
                        for j in range(1, D // 16):
                            p += (ebuf[erow, pl.ds(eoff + j * 16, 16)] *
                                  ubuf[li, pl.ds(j * 16, 16)])
                        for perm in perms:
                            p = p + p.at[perm].get(mode="promise_in_bounds")
                        acc = jnp.where(lane == t, p, acc)
                    plsc.store_scatter(wide, [g * 16 + lane, zero16], acc)
                    return carry

                lax.fori_loop(0, 8, group, 0)
                pltpu.async_copy(
                    wide,
                    out.at[plsc.Indices(mbs2d.at[k], ignored_value=-1)],
                    semg).wait()

    return body


def _make_kernel(is_item):
    mesh = plsc.VectorSubcoreMesh(core_axis_name="c", subcore_axis_name="s")
    scratch = [
        pltpu.VMEM((16, 128), jnp.int32),
        pltpu.VMEM((D, 128), jnp.float32),
        pltpu.VMEM((D, 128), jnp.float32),
        pltpu.VMEM((D, 128), jnp.float32),
        pltpu.VMEM((D, 128), jnp.float32),
        pltpu.VMEM((MCAP,), jnp.int32),
        pltpu.VMEM((MCAP,), jnp.int32),
        pltpu.VMEM((NCHK, 128), jnp.int32),
        pltpu.VMEM((16, 256), jnp.int32),
        pltpu.VMEM((16, 256), jnp.int32),
        pltpu.VMEM((272,), jnp.int32),
        pltpu.VMEM((272,), jnp.int32),
        pltpu.VMEM((MCAP,), jnp.int32),
        pltpu.VMEM((MCAP,), jnp.int32),
        pltpu.VMEM((MCAP // 2, 2 * D), jnp.float32),
    ]
    if is_item:
        scratch += [pltpu.VMEM((128, 128), jnp.float32)]
    scratch += [pltpu.VMEM((128, 128), jnp.float32)]
    scratch += [pltpu.SemaphoreType.DMA] * 5
    out_type = jax.ShapeDtypeStruct((B, 128), jnp.float32)
    return functools.partial(
        pl.kernel, mesh=mesh, out_type=out_type, scratch_types=scratch,
        compiler_params=pltpu.CompilerParams(needs_layout_passes=False),
    )(_stream_pass(is_item))


@jax.jit
def kernel(user_ids, item_ids, user_table, item_table):
    ut_t = user_table.T
    it_t = item_table.T
    uids2d = user_ids.reshape(B // 128, 128)
    iids2d = item_ids.reshape(B // 128, 128)
    staging = _make_kernel(False)(ut_t, uids2d)
    out2d = _make_kernel(True)(it_t, iids2d, staging)
    return out2d[:, 0]
